# baseline (speedup 1.0000x reference)
import functools

import jax
import jax.numpy as jnp
from jax import lax
from jax.experimental import pallas as pl
from jax.experimental.pallas import tpu as pltpu
from jax.experimental.pallas import tpu_sc as plsc

N = 10000
E = 320000
D = 128
A = 16
T = 16

NC = 2
NS = 16
NW = NC * NS

CH = 80
NCHUNKS = E // CH
KPT = NCHUNKS // NW
ROWS_A = 624
TAIL_ROWS = N - NS * ROWS_A
ZROWS = 48


def _proj_nodes_body(fv_ref, w1_ref, p_ref):
    p_ref[...] = jnp.dot(fv_ref[...], w1_ref[...],
                         preferred_element_type=jnp.float32)


def _proj_edges_body(ea_ref, et_ref, w2a_ref, w2b_ref, b_ref, u_ref):
    u_ref[...] = (
        jnp.dot(ea_ref[...], w2a_ref[...], preferred_element_type=jnp.float32)
        + jnp.dot(et_ref[...], w2b_ref[...], preferred_element_type=jnp.float32)
        + b_ref[...]
    )


def _sc_scatter_body(p_hbm, u_hbm, src_hbm, dst_hbm, out_hbm,
                     src_v, dst_v, rows_v, u_v, zero_v, acc_sh,
                     sem_in0, sem_in1, sem_g0, sem_g1, sem_z):
    cid = lax.axis_index("c")
    sid = lax.axis_index("s")
    wid = sid * NC + cid
    sem_in = (sem_in0, sem_in1)
    sem_g = (sem_g0, sem_g1)

    def zero_buf(i, _):
        r = i // (D // 16)
        c = (i % (D // 16)) * 16
        zero_v[r, pl.ds(c, 16)] = jnp.zeros((16,), jnp.float32)
        return 0
    lax.fori_loop(0, ZROWS * (D // 16), zero_buf, 0, unroll=8)
    row0 = sid * ROWS_A
    zcopies = []
    for z in range(ROWS_A // ZROWS):
        zcopies.append(pltpu.async_copy(
            zero_v, acc_sh.at[pl.ds(row0 + z * ZROWS, ZROWS)], sem_z))

    @pl.when(sid == NS - 1)
    def _zero_tail():
        pltpu.async_copy(zero_v.at[pl.ds(0, TAIL_ROWS)],
                         acc_sh.at[pl.ds(NS * ROWS_A, TAIL_ROWS)], sem_z).wait()
    for zc in zcopies:
        zc.wait()
    plsc.subcore_barrier()

    def issue_in(k, b):
        base = (k * NW + wid) * CH
        pltpu.async_copy(src_hbm.at[pl.ds(base, CH)], src_v.at[b], sem_in[b])
        pltpu.async_copy(dst_hbm.at[pl.ds(base, CH)], dst_v.at[b], sem_in[b])
        pltpu.async_copy(u_hbm.at[pl.ds(base, CH)], u_v.at[b], sem_in[b])

    def wait_in_idx(b):
        pltpu.make_async_copy(src_hbm.at[pl.ds(0, CH)], src_v.at[b],
                              sem_in[b]).wait()
        pltpu.make_async_copy(dst_hbm.at[pl.ds(0, CH)], dst_v.at[b],
                              sem_in[b]).wait()

    def wait_in_u(b):
        pltpu.make_async_copy(u_hbm.at[pl.ds(0, CH)], u_v.at[b],
                              sem_in[b]).wait()

    def issue_gather(b):
        pltpu.async_copy(p_hbm.at[src_v.at[b]], rows_v.at[b], sem_g[b])

    def wait_gather(b):
        pltpu.make_async_copy(p_hbm.at[src_v.at[b]], rows_v.at[b],
                              sem_g[b]).wait()

    issue_in(0, 0)
    wait_in_idx(0)
    issue_gather(0)
    issue_in(1, 1)

    def phase(k, cur, nxt):
        wait_gather(cur)
        wait_in_u(cur)

        @pl.when(k + 1 < KPT)
        def _prefetch_gather():
            wait_in_idx(nxt)
            issue_gather(nxt)

        def fuse(r, _):
            for c in range(D // 16):
                v = (rows_v[cur, r, pl.ds(c * 16, 16)]
                     + u_v[cur, r, pl.ds(c * 16, 16)])
                rows_v[cur, r, pl.ds(c * 16, 16)] = jnp.maximum(v, 0.0)
            return 0
        lax.fori_loop(0, CH, fuse, 0, unroll=2)

        pltpu.sync_copy(rows_v.at[cur], acc_sh.at[dst_v.at[cur]], add=True)

        @pl.when(k + 2 < KPT)
        def _prefetch_in():
            issue_in(k + 2, cur)

    def do_chunk(k, _):
        @pl.when(k % 2 == 0)
        def _even():
            phase(k, 0, 1)

        @pl.when(k % 2 == 1)
        def _odd():
            phase(k, 1, 0)
        return 0
    lax.fori_loop(0, KPT, do_chunk, 0)

    plsc.subcore_barrier()

    pltpu.sync_copy(acc_sh.at[pl.ds(row0, ROWS_A)],
                    out_hbm.at[pl.ds(cid * N + row0, ROWS_A)])

    @pl.when(sid == NS - 1)
    def _drain_tail():
        pltpu.sync_copy(acc_sh.at[pl.ds(NS * ROWS_A, TAIL_ROWS)],
                        out_hbm.at[pl.ds(cid * N + NS * ROWS_A, TAIL_ROWS)])


def _final_body(a0_ref, a1_ref, bc_ref, wl_ref, bl_ref, g_ref, be_ref, o_ref):
    h = a0_ref[...] + a1_ref[...] + bc_ref[...]
    y = jnp.dot(h, wl_ref[...], preferred_element_type=jnp.float32) + bl_ref[...]
    mean = jnp.mean(y, axis=-1, keepdims=True)
    var = jnp.mean(jnp.square(y - mean), axis=-1, keepdims=True)
    yn = (y - mean) * lax.rsqrt(var + 1e-5) * g_ref[...] + be_ref[...]
    o_ref[...] = jnp.maximum(yn, 0.0)


def kernel(feature_view, edge_index, edge_attr, edge_time_emb,
           boundary_condition, W_msg, b_msg, W_lin, b_lin, ln_gamma, ln_beta):
    src = edge_index[0]
    dst = edge_index[1]
    w1 = W_msg[:D]
    w2a = W_msg[D:D + A]
    w2b = W_msg[D + A:]
    b2 = b_msg.reshape(1, D)

    BN = 1000
    p = pl.pallas_call(
        _proj_nodes_body,
        grid=(N // BN,),
        in_specs=[
            pl.BlockSpec((BN, D), lambda i: (i, 0)),
            pl.BlockSpec((D, D), lambda i: (0, 0)),
        ],
        out_specs=pl.BlockSpec((BN, D), lambda i: (i, 0)),
        out_shape=jax.ShapeDtypeStruct((N, D), jnp.float32),
    )(feature_view, w1)

    BE = 4000
    u = pl.pallas_call(
        _proj_edges_body,
        grid=(E // BE,),
        in_specs=[
            pl.BlockSpec((BE, A), lambda i: (i, 0)),
            pl.BlockSpec((BE, T), lambda i: (i, 0)),
            pl.BlockSpec((A, D), lambda i: (0, 0)),
            pl.BlockSpec((T, D), lambda i: (0, 0)),
            pl.BlockSpec((1, D), lambda i: (0, 0)),
        ],
        out_specs=pl.BlockSpec((BE, D), lambda i: (i, 0)),
        out_shape=jax.ShapeDtypeStruct((E, D), jnp.float32),
    )(edge_attr, edge_time_emb, w2a, w2b, b2)

    mesh = plsc.VectorSubcoreMesh(core_axis_name="c", subcore_axis_name="s",
                                  num_cores=NC, num_subcores=NS)
    acc2 = pl.kernel(
        _sc_scatter_body,
        out_type=jax.ShapeDtypeStruct((NC * N, D), jnp.float32),
        mesh=mesh,
        scratch_types=[
            pltpu.VMEM((2, CH), jnp.int32),
            pltpu.VMEM((2, CH), jnp.int32),
            pltpu.VMEM((2, CH, D), jnp.float32),
            pltpu.VMEM((2, CH, D), jnp.float32),
            pltpu.VMEM((ZROWS, D), jnp.float32),
            pltpu.VMEM_SHARED((N, D), jnp.float32),
            pltpu.SemaphoreType.DMA,
            pltpu.SemaphoreType.DMA,
            pltpu.SemaphoreType.DMA,
            pltpu.SemaphoreType.DMA,
            pltpu.SemaphoreType.DMA,
        ],
    )(p, u, src, dst)

    out = pl.pallas_call(
        _final_body,
        grid=(N // BN,),
        in_specs=[
            pl.BlockSpec((BN, D), lambda i: (i, 0)),
            pl.BlockSpec((BN, D), lambda i: (i + N // BN, 0)),
            pl.BlockSpec((BN, D), lambda i: (i, 0)),
            pl.BlockSpec((D, D), lambda i: (0, 0)),
            pl.BlockSpec((1, D), lambda i: (0, 0)),
            pl.BlockSpec((1, D), lambda i: (0, 0)),
            pl.BlockSpec((1, D), lambda i: (0, 0)),
        ],
        out_specs=pl.BlockSpec((BN, D), lambda i: (i, 0)),
        out_shape=jax.ShapeDtypeStruct((N, D), jnp.float32),
    )(acc2, acc2, boundary_condition, W_lin, b_lin.reshape(1, D),
      ln_gamma.reshape(1, D), ln_beta.reshape(1, D))

    return out


# speedup vs baseline: 1.5267x; 1.5267x over previous
import functools

import jax
import jax.numpy as jnp
from jax import lax
from jax.experimental import pallas as pl
from jax.experimental.pallas import tpu as pltpu
from jax.experimental.pallas import tpu_sc as plsc

N = 10000
E = 320000
D = 128
A = 16
T = 16

NC = 2
NS = 16
NW = NC * NS

CH = 80
NCHUNKS = E // CH
KPT = NCHUNKS // NW
ROWS_A = 624
TAIL_ROWS = N - NS * ROWS_A
ZROWS = 48


def _proj_nodes_body(fv_ref, w1_ref, p_ref):
    p_ref[...] = jnp.dot(fv_ref[...], w1_ref[...],
                         preferred_element_type=jnp.float32)


def _proj_edges_body(ea_ref, et_ref, w2a_ref, w2b_ref, b_ref, u_ref):
    u_ref[...] = (
        jnp.dot(ea_ref[...], w2a_ref[...], preferred_element_type=jnp.float32)
        + jnp.dot(et_ref[...], w2b_ref[...], preferred_element_type=jnp.float32)
        + b_ref[...]
    )


def _sc_scatter_body(p_hbm, u_hbm, src_hbm, dst_hbm, out_hbm,
                     src_v, dst_v, rows_v, u_v, zero_v, acc_sh,
                     sem_in0, sem_in1, sem_g0, sem_g1, sem_z):
    cid = lax.axis_index("c")
    sid = lax.axis_index("s")
    wid = sid * NC + cid
    sem_in = (sem_in0, sem_in1)
    sem_g = (sem_g0, sem_g1)

    def zero_buf(i, _):
        r = i // (D // 16)
        c = (i % (D // 16)) * 16
        zero_v[r, pl.ds(c, 16)] = jnp.zeros((16,), jnp.float32)
        return 0
    lax.fori_loop(0, ZROWS * (D // 16), zero_buf, 0, unroll=8)
    row0 = sid * ROWS_A
    zcopies = []
    for z in range(ROWS_A // ZROWS):
        zcopies.append(pltpu.async_copy(
            zero_v, acc_sh.at[pl.ds(row0 + z * ZROWS, ZROWS)], sem_z))

    @pl.when(sid == NS - 1)
    def _zero_tail():
        pltpu.async_copy(zero_v.at[pl.ds(0, TAIL_ROWS)],
                         acc_sh.at[pl.ds(NS * ROWS_A, TAIL_ROWS)], sem_z).wait()
    for zc in zcopies:
        zc.wait()
    plsc.subcore_barrier()

    def issue_in(k, b):
        base = (k * NW + wid) * CH
        pltpu.async_copy(src_hbm.at[pl.ds(base, CH)], src_v.at[b], sem_in[b])
        pltpu.async_copy(dst_hbm.at[pl.ds(base, CH)], dst_v.at[b], sem_in[b])
        pltpu.async_copy(u_hbm.at[pl.ds(base, CH)], u_v.at[b], sem_in[b])

    def wait_in_idx(b):
        pltpu.make_async_copy(src_hbm.at[pl.ds(0, CH)], src_v.at[b],
                              sem_in[b]).wait()
        pltpu.make_async_copy(dst_hbm.at[pl.ds(0, CH)], dst_v.at[b],
                              sem_in[b]).wait()

    def wait_in_u(b):
        pltpu.make_async_copy(u_hbm.at[pl.ds(0, CH)], u_v.at[b],
                              sem_in[b]).wait()

    def issue_gather(b):
        pltpu.async_copy(p_hbm.at[src_v.at[b]], rows_v.at[b], sem_g[b])

    def wait_gather(b):
        pltpu.make_async_copy(p_hbm.at[src_v.at[b]], rows_v.at[b],
                              sem_g[b]).wait()

    issue_in(0, 0)
    wait_in_idx(0)
    issue_gather(0)
    issue_in(1, 1)

    def phase(k, cur, nxt):
        wait_gather(cur)
        wait_in_u(cur)

        @pl.when(k + 1 < KPT)
        def _prefetch_gather():
            wait_in_idx(nxt)
            issue_gather(nxt)

        def fuse(r, _):
            for c in range(D // 16):
                v = (rows_v[cur, r, pl.ds(c * 16, 16)]
                     + u_v[cur, r, pl.ds(c * 16, 16)])
                rows_v[cur, r, pl.ds(c * 16, 16)] = jnp.maximum(v, 0.0)
            return 0
        lax.fori_loop(0, CH, fuse, 0)

        pltpu.sync_copy(rows_v.at[cur], acc_sh.at[dst_v.at[cur]], add=True)

        @pl.when(k + 2 < KPT)
        def _prefetch_in():
            issue_in(k + 2, cur)

    def do_pair(k2, _):
        phase(2 * k2, 0, 1)
        phase(2 * k2 + 1, 1, 0)
        return 0
    lax.fori_loop(0, KPT // 2, do_pair, 0)
    phase(jnp.int32(KPT - 1), 0, 1)

    plsc.subcore_barrier()

    pltpu.sync_copy(acc_sh.at[pl.ds(row0, ROWS_A)],
                    out_hbm.at[pl.ds(cid * N + row0, ROWS_A)])

    @pl.when(sid == NS - 1)
    def _drain_tail():
        pltpu.sync_copy(acc_sh.at[pl.ds(NS * ROWS_A, TAIL_ROWS)],
                        out_hbm.at[pl.ds(cid * N + NS * ROWS_A, TAIL_ROWS)])


def _final_body(a0_ref, a1_ref, bc_ref, wl_ref, bl_ref, g_ref, be_ref, o_ref):
    h = a0_ref[...] + a1_ref[...] + bc_ref[...]
    y = jnp.dot(h, wl_ref[...], preferred_element_type=jnp.float32) + bl_ref[...]
    mean = jnp.mean(y, axis=-1, keepdims=True)
    var = jnp.mean(jnp.square(y - mean), axis=-1, keepdims=True)
    yn = (y - mean) * lax.rsqrt(var + 1e-5) * g_ref[...] + be_ref[...]
    o_ref[...] = jnp.maximum(yn, 0.0)


def kernel(feature_view, edge_index, edge_attr, edge_time_emb,
           boundary_condition, W_msg, b_msg, W_lin, b_lin, ln_gamma, ln_beta):
    src = edge_index[0]
    dst = edge_index[1]
    w1 = W_msg[:D]
    w2a = W_msg[D:D + A]
    w2b = W_msg[D + A:]
    b2 = b_msg.reshape(1, D)

    BN = 1000
    p = pl.pallas_call(
        _proj_nodes_body,
        grid=(N // BN,),
        in_specs=[
            pl.BlockSpec((BN, D), lambda i: (i, 0)),
            pl.BlockSpec((D, D), lambda i: (0, 0)),
        ],
        out_specs=pl.BlockSpec((BN, D), lambda i: (i, 0)),
        out_shape=jax.ShapeDtypeStruct((N, D), jnp.float32),
    )(feature_view, w1)

    BE = 4000
    u = pl.pallas_call(
        _proj_edges_body,
        grid=(E // BE,),
        in_specs=[
            pl.BlockSpec((BE, A), lambda i: (i, 0)),
            pl.BlockSpec((BE, T), lambda i: (i, 0)),
            pl.BlockSpec((A, D), lambda i: (0, 0)),
            pl.BlockSpec((T, D), lambda i: (0, 0)),
            pl.BlockSpec((1, D), lambda i: (0, 0)),
        ],
        out_specs=pl.BlockSpec((BE, D), lambda i: (i, 0)),
        out_shape=jax.ShapeDtypeStruct((E, D), jnp.float32),
    )(edge_attr, edge_time_emb, w2a, w2b, b2)

    mesh = plsc.VectorSubcoreMesh(core_axis_name="c", subcore_axis_name="s",
                                  num_cores=NC, num_subcores=NS)
    acc2 = pl.kernel(
        _sc_scatter_body,
        out_type=jax.ShapeDtypeStruct((NC * N, D), jnp.float32),
        mesh=mesh,
        scratch_types=[
            pltpu.VMEM((2, CH), jnp.int32),
            pltpu.VMEM((2, CH), jnp.int32),
            pltpu.VMEM((2, CH, D), jnp.float32),
            pltpu.VMEM((2, CH, D), jnp.float32),
            pltpu.VMEM((ZROWS, D), jnp.float32),
            pltpu.VMEM_SHARED((N, D), jnp.float32),
            pltpu.SemaphoreType.DMA,
            pltpu.SemaphoreType.DMA,
            pltpu.SemaphoreType.DMA,
            pltpu.SemaphoreType.DMA,
            pltpu.SemaphoreType.DMA,
        ],
    )(p, u, src, dst)

    out = pl.pallas_call(
        _final_body,
        grid=(N // BN,),
        in_specs=[
            pl.BlockSpec((BN, D), lambda i: (i, 0)),
            pl.BlockSpec((BN, D), lambda i: (i + N // BN, 0)),
            pl.BlockSpec((BN, D), lambda i: (i, 0)),
            pl.BlockSpec((D, D), lambda i: (0, 0)),
            pl.BlockSpec((1, D), lambda i: (0, 0)),
            pl.BlockSpec((1, D), lambda i: (0, 0)),
            pl.BlockSpec((1, D), lambda i: (0, 0)),
        ],
        out_specs=pl.BlockSpec((BN, D), lambda i: (i, 0)),
        out_shape=jax.ShapeDtypeStruct((N, D), jnp.float32),
    )(acc2, acc2, boundary_condition, W_lin, b_lin.reshape(1, D),
      ln_gamma.reshape(1, D), ln_beta.reshape(1, D))

    return out


# speedup vs baseline: 1.5350x; 1.0055x over previous
import functools

import jax
import jax.numpy as jnp
from jax import lax
from jax.experimental import pallas as pl
from jax.experimental.pallas import tpu as pltpu
from jax.experimental.pallas import tpu_sc as plsc

N = 10000
E = 320000
D = 128
A = 16
T = 16

NC = 2
NS = 16
NW = NC * NS

CH = 80
NCHUNKS = E // CH
KPT = NCHUNKS // NW
ROWS_A = 624
TAIL_ROWS = N - NS * ROWS_A
ZROWS = 48


def _proj_nodes_body(fv_ref, w1_ref, p_ref):
    p_ref[...] = jnp.dot(fv_ref[...], w1_ref[...],
                         preferred_element_type=jnp.float32)


def _proj_edges_body(ea_ref, et_ref, w2a_ref, w2b_ref, b_ref, u_ref):
    u = (
        jnp.dot(ea_ref[...], w2a_ref[...], preferred_element_type=jnp.float32)
        + jnp.dot(et_ref[...], w2b_ref[...], preferred_element_type=jnp.float32)
        + b_ref[...]
    )
    lo = lax.bitcast_convert_type(u[:, :D // 2].astype(jnp.bfloat16),
                                  jnp.uint16).astype(jnp.uint32)
    hi = lax.bitcast_convert_type(u[:, D // 2:].astype(jnp.bfloat16),
                                  jnp.uint16).astype(jnp.uint32)
    u_ref[...] = lax.bitcast_convert_type((hi << 16) | lo, jnp.int32)


def _sc_scatter_body(p_hbm, u_hbm, src_hbm, dst_hbm, out_hbm,
                     src_v, dst_v, rows_v, u_v, zero_v, acc_sh,
                     sem_in0, sem_in1, sem_g0, sem_g1, sem_z):
    cid = lax.axis_index("c")
    sid = lax.axis_index("s")
    wid = sid * NC + cid
    sem_in = (sem_in0, sem_in1)
    sem_g = (sem_g0, sem_g1)

    def zero_buf(i, _):
        r = i // (D // 16)
        c = (i % (D // 16)) * 16
        zero_v[r, pl.ds(c, 16)] = jnp.zeros((16,), jnp.float32)
        return 0
    lax.fori_loop(0, ZROWS * (D // 16), zero_buf, 0, unroll=8)
    row0 = sid * ROWS_A
    zcopies = []
    for z in range(ROWS_A // ZROWS):
        zcopies.append(pltpu.async_copy(
            zero_v, acc_sh.at[pl.ds(row0 + z * ZROWS, ZROWS)], sem_z))

    @pl.when(sid == NS - 1)
    def _zero_tail():
        pltpu.async_copy(zero_v.at[pl.ds(0, TAIL_ROWS)],
                         acc_sh.at[pl.ds(NS * ROWS_A, TAIL_ROWS)], sem_z).wait()
    for zc in zcopies:
        zc.wait()
    plsc.subcore_barrier()

    def issue_in(k, b):
        base = (k * NW + wid) * CH
        pltpu.async_copy(src_hbm.at[pl.ds(base, CH)], src_v.at[b], sem_in[b])
        pltpu.async_copy(dst_hbm.at[pl.ds(base, CH)], dst_v.at[b], sem_in[b])
        pltpu.async_copy(u_hbm.at[pl.ds(base, CH)], u_v.at[b], sem_in[b])

    def wait_in_idx(b):
        pltpu.make_async_copy(src_hbm.at[pl.ds(0, CH)], src_v.at[b],
                              sem_in[b]).wait()
        pltpu.make_async_copy(dst_hbm.at[pl.ds(0, CH)], dst_v.at[b],
                              sem_in[b]).wait()

    def wait_in_u(b):
        pltpu.make_async_copy(u_hbm.at[pl.ds(0, CH)], u_v.at[b],
                              sem_in[b]).wait()

    def issue_gather(b):
        pltpu.async_copy(p_hbm.at[src_v.at[b]], rows_v.at[b], sem_g[b])

    def wait_gather(b):
        pltpu.make_async_copy(p_hbm.at[src_v.at[b]], rows_v.at[b],
                              sem_g[b]).wait()

    issue_in(0, 0)
    wait_in_idx(0)
    issue_gather(0)
    issue_in(1, 1)

    def phase(k, cur, nxt):
        wait_gather(cur)
        wait_in_u(cur)

        @pl.when(k + 1 < KPT)
        def _prefetch_gather():
            wait_in_idx(nxt)
            issue_gather(nxt)

        def fuse(r, _):
            for m in range(D // 32):
                w = u_v[cur, r, pl.ds(m * 16, 16)]
                wl = lax.bitcast_convert_type(lax.shift_left(w, 16),
                                              jnp.float32)
                wh = lax.bitcast_convert_type(
                    jnp.bitwise_and(w, jnp.int32(-65536)), jnp.float32)
                a = rows_v[cur, r, pl.ds(m * 16, 16)] + wl
                rows_v[cur, r, pl.ds(m * 16, 16)] = jnp.maximum(a, 0.0)
                b = rows_v[cur, r, pl.ds((m + D // 32) * 16, 16)] + wh
                rows_v[cur, r, pl.ds((m + D // 32) * 16, 16)] = (
                    jnp.maximum(b, 0.0))
            return 0
        lax.fori_loop(0, CH, fuse, 0)

        pltpu.sync_copy(rows_v.at[cur], acc_sh.at[dst_v.at[cur]], add=True)

        @pl.when(k + 2 < KPT)
        def _prefetch_in():
            issue_in(k + 2, cur)

    def do_pair(k2, _):
        phase(2 * k2, 0, 1)
        phase(2 * k2 + 1, 1, 0)
        return 0
    lax.fori_loop(0, KPT // 2, do_pair, 0)
    phase(jnp.int32(KPT - 1), 0, 1)

    plsc.subcore_barrier()

    pltpu.sync_copy(acc_sh.at[pl.ds(row0, ROWS_A)],
                    out_hbm.at[pl.ds(cid * N + row0, ROWS_A)])

    @pl.when(sid == NS - 1)
    def _drain_tail():
        pltpu.sync_copy(acc_sh.at[pl.ds(NS * ROWS_A, TAIL_ROWS)],
                        out_hbm.at[pl.ds(cid * N + NS * ROWS_A, TAIL_ROWS)])


def _final_body(a0_ref, a1_ref, bc_ref, wl_ref, bl_ref, g_ref, be_ref, o_ref):
    h = a0_ref[...] + a1_ref[...] + bc_ref[...]
    y = jnp.dot(h, wl_ref[...], preferred_element_type=jnp.float32) + bl_ref[...]
    mean = jnp.mean(y, axis=-1, keepdims=True)
    var = jnp.mean(jnp.square(y - mean), axis=-1, keepdims=True)
    yn = (y - mean) * lax.rsqrt(var + 1e-5) * g_ref[...] + be_ref[...]
    o_ref[...] = jnp.maximum(yn, 0.0)


def kernel(feature_view, edge_index, edge_attr, edge_time_emb,
           boundary_condition, W_msg, b_msg, W_lin, b_lin, ln_gamma, ln_beta):
    src = edge_index[0]
    dst = edge_index[1]
    w1 = W_msg[:D]
    w2a = W_msg[D:D + A]
    w2b = W_msg[D + A:]
    b2 = b_msg.reshape(1, D)

    BN = 1000
    p = pl.pallas_call(
        _proj_nodes_body,
        grid=(N // BN,),
        in_specs=[
            pl.BlockSpec((BN, D), lambda i: (i, 0)),
            pl.BlockSpec((D, D), lambda i: (0, 0)),
        ],
        out_specs=pl.BlockSpec((BN, D), lambda i: (i, 0)),
        out_shape=jax.ShapeDtypeStruct((N, D), jnp.float32),
    )(feature_view, w1)

    BE = 4000
    u = pl.pallas_call(
        _proj_edges_body,
        grid=(E // BE,),
        in_specs=[
            pl.BlockSpec((BE, A), lambda i: (i, 0)),
            pl.BlockSpec((BE, T), lambda i: (i, 0)),
            pl.BlockSpec((A, D), lambda i: (0, 0)),
            pl.BlockSpec((T, D), lambda i: (0, 0)),
            pl.BlockSpec((1, D), lambda i: (0, 0)),
        ],
        out_specs=pl.BlockSpec((BE, D // 2), lambda i: (i, 0)),
        out_shape=jax.ShapeDtypeStruct((E, D // 2), jnp.int32),
    )(edge_attr, edge_time_emb, w2a, w2b, b2)

    mesh = plsc.VectorSubcoreMesh(core_axis_name="c", subcore_axis_name="s",
                                  num_cores=NC, num_subcores=NS)
    acc2 = pl.kernel(
        _sc_scatter_body,
        out_type=jax.ShapeDtypeStruct((NC * N, D), jnp.float32),
        mesh=mesh,
        scratch_types=[
            pltpu.VMEM((2, CH), jnp.int32),
            pltpu.VMEM((2, CH), jnp.int32),
            pltpu.VMEM((2, CH, D), jnp.float32),
            pltpu.VMEM((2, CH, D // 2), jnp.int32),
            pltpu.VMEM((ZROWS, D), jnp.float32),
            pltpu.VMEM_SHARED((N, D), jnp.float32),
            pltpu.SemaphoreType.DMA,
            pltpu.SemaphoreType.DMA,
            pltpu.SemaphoreType.DMA,
            pltpu.SemaphoreType.DMA,
            pltpu.SemaphoreType.DMA,
        ],
    )(p, u, src, dst)

    out = pl.pallas_call(
        _final_body,
        grid=(N // BN,),
        in_specs=[
            pl.BlockSpec((BN, D), lambda i: (i, 0)),
            pl.BlockSpec((BN, D), lambda i: (i + N // BN, 0)),
            pl.BlockSpec((BN, D), lambda i: (i, 0)),
            pl.BlockSpec((D, D), lambda i: (0, 0)),
            pl.BlockSpec((1, D), lambda i: (0, 0)),
            pl.BlockSpec((1, D), lambda i: (0, 0)),
            pl.BlockSpec((1, D), lambda i: (0, 0)),
        ],
        out_specs=pl.BlockSpec((BN, D), lambda i: (i, 0)),
        out_shape=jax.ShapeDtypeStruct((N, D), jnp.float32),
    )(acc2, acc2, boundary_condition, W_lin, b_lin.reshape(1, D),
      ln_gamma.reshape(1, D), ln_beta.reshape(1, D))

    return out


# speedup vs baseline: 2.1231x; 1.3831x over previous
import functools

import jax
import jax.numpy as jnp
from jax import lax
from jax.experimental import pallas as pl
from jax.experimental.pallas import tpu as pltpu
from jax.experimental.pallas import tpu_sc as plsc

N = 10000
E = 320000
D = 128
A = 16
T = 16

NC = 2
NS = 16
NW = NC * NS

CH = 80
NCHUNKS = E // CH
KPT = NCHUNKS // NW
ROWS_A = 624
TAIL_ROWS = N - NS * ROWS_A
ZROWS = 48


def _proj_nodes_body(fv_ref, w1_ref, p_ref):
    p_ref[...] = jnp.dot(fv_ref[...], w1_ref[...],
                         preferred_element_type=jnp.float32)


def _proj_edges_body(ea_ref, et_ref, w2a_ref, w2b_ref, b_ref, u_ref):
    dn = (((0,), (0,)), ((), ()))
    u = (
        lax.dot_general(ea_ref[...], w2a_ref[...], dn,
                        preferred_element_type=jnp.float32)
        + lax.dot_general(et_ref[...], w2b_ref[...], dn,
                          preferred_element_type=jnp.float32)
        + b_ref[...]
    )
    lo = lax.bitcast_convert_type(u[:, :D // 2].astype(jnp.bfloat16),
                                  jnp.uint16).astype(jnp.uint32)
    hi = lax.bitcast_convert_type(u[:, D // 2:].astype(jnp.bfloat16),
                                  jnp.uint16).astype(jnp.uint32)
    u_ref[...] = lax.bitcast_convert_type((hi << 16) | lo, jnp.int32)


def _sc_scatter_body(p_hbm, u_hbm, src_hbm, dst_hbm, out_hbm,
                     src_v, dst_v, rows_v, u_v, zero_v, acc_sh,
                     sem_in0, sem_in1, sem_g0, sem_g1, sem_z):
    cid = lax.axis_index("c")
    sid = lax.axis_index("s")
    wid = sid * NC + cid
    sem_in = (sem_in0, sem_in1)
    sem_g = (sem_g0, sem_g1)

    def zero_buf(i, _):
        r = i // (D // 16)
        c = (i % (D // 16)) * 16
        zero_v[r, pl.ds(c, 16)] = jnp.zeros((16,), jnp.float32)
        return 0
    lax.fori_loop(0, ZROWS * (D // 16), zero_buf, 0, unroll=8)
    row0 = sid * ROWS_A
    zcopies = []
    for z in range(ROWS_A // ZROWS):
        zcopies.append(pltpu.async_copy(
            zero_v, acc_sh.at[pl.ds(row0 + z * ZROWS, ZROWS)], sem_z))

    @pl.when(sid == NS - 1)
    def _zero_tail():
        pltpu.async_copy(zero_v.at[pl.ds(0, TAIL_ROWS)],
                         acc_sh.at[pl.ds(NS * ROWS_A, TAIL_ROWS)], sem_z).wait()
    for zc in zcopies:
        zc.wait()
    plsc.subcore_barrier()

    def issue_in(k, b):
        base = (k * NW + wid) * CH
        pltpu.async_copy(src_hbm.at[pl.ds(base, CH)], src_v.at[b], sem_in[b])
        pltpu.async_copy(dst_hbm.at[pl.ds(base, CH)], dst_v.at[b], sem_in[b])
        pltpu.async_copy(u_hbm.at[pl.ds(base, CH)], u_v.at[b], sem_in[b])

    def wait_in_idx(b):
        pltpu.make_async_copy(src_hbm.at[pl.ds(0, CH)], src_v.at[b],
                              sem_in[b]).wait()
        pltpu.make_async_copy(dst_hbm.at[pl.ds(0, CH)], dst_v.at[b],
                              sem_in[b]).wait()

    def wait_in_u(b):
        pltpu.make_async_copy(u_hbm.at[pl.ds(0, CH)], u_v.at[b],
                              sem_in[b]).wait()

    def issue_gather(b):
        pltpu.async_copy(p_hbm.at[src_v.at[b]], rows_v.at[b], sem_g[b])

    def wait_gather(b):
        pltpu.make_async_copy(p_hbm.at[src_v.at[b]], rows_v.at[b],
                              sem_g[b]).wait()

    issue_in(0, 0)
    wait_in_idx(0)
    issue_gather(0)
    issue_in(1, 1)

    def phase(k, cur, nxt):
        wait_gather(cur)
        wait_in_u(cur)

        @pl.when(k + 1 < KPT)
        def _prefetch_gather():
            wait_in_idx(nxt)
            issue_gather(nxt)

        def fuse(r, _):
            for m in range(D // 32):
                w = u_v[cur, r, pl.ds(m * 16, 16)]
                wl = lax.bitcast_convert_type(lax.shift_left(w, 16),
                                              jnp.float32)
                wh = lax.bitcast_convert_type(
                    jnp.bitwise_and(w, jnp.int32(-65536)), jnp.float32)
                a = rows_v[cur, r, pl.ds(m * 16, 16)] + wl
                rows_v[cur, r, pl.ds(m * 16, 16)] = jnp.maximum(a, 0.0)
                b = rows_v[cur, r, pl.ds((m + D // 32) * 16, 16)] + wh
                rows_v[cur, r, pl.ds((m + D // 32) * 16, 16)] = (
                    jnp.maximum(b, 0.0))
            return 0
        lax.fori_loop(0, CH, fuse, 0)

        pltpu.sync_copy(rows_v.at[cur], acc_sh.at[dst_v.at[cur]], add=True)

        @pl.when(k + 2 < KPT)
        def _prefetch_in():
            issue_in(k + 2, cur)

    def do_pair(k2, _):
        phase(2 * k2, 0, 1)
        phase(2 * k2 + 1, 1, 0)
        return 0
    lax.fori_loop(0, KPT // 2, do_pair, 0)
    phase(jnp.int32(KPT - 1), 0, 1)

    plsc.subcore_barrier()

    pltpu.sync_copy(acc_sh.at[pl.ds(row0, ROWS_A)],
                    out_hbm.at[pl.ds(cid * N + row0, ROWS_A)])

    @pl.when(sid == NS - 1)
    def _drain_tail():
        pltpu.sync_copy(acc_sh.at[pl.ds(NS * ROWS_A, TAIL_ROWS)],
                        out_hbm.at[pl.ds(cid * N + NS * ROWS_A, TAIL_ROWS)])


def _final_body(a0_ref, a1_ref, bc_ref, wl_ref, bl_ref, g_ref, be_ref, o_ref):
    h = a0_ref[...] + a1_ref[...] + bc_ref[...]
    y = jnp.dot(h, wl_ref[...], preferred_element_type=jnp.float32) + bl_ref[...]
    mean = jnp.mean(y, axis=-1, keepdims=True)
    var = jnp.mean(jnp.square(y - mean), axis=-1, keepdims=True)
    yn = (y - mean) * lax.rsqrt(var + 1e-5) * g_ref[...] + be_ref[...]
    o_ref[...] = jnp.maximum(yn, 0.0)


def kernel(feature_view, edge_index, edge_attr, edge_time_emb,
           boundary_condition, W_msg, b_msg, W_lin, b_lin, ln_gamma, ln_beta):
    src = edge_index[0]
    dst = edge_index[1]
    w1 = W_msg[:D]
    w2a = W_msg[D:D + A]
    w2b = W_msg[D + A:]
    b2 = b_msg.reshape(1, D)

    BN = 1000
    p = pl.pallas_call(
        _proj_nodes_body,
        grid=(N // BN,),
        in_specs=[
            pl.BlockSpec((BN, D), lambda i: (i, 0)),
            pl.BlockSpec((D, D), lambda i: (0, 0)),
        ],
        out_specs=pl.BlockSpec((BN, D), lambda i: (i, 0)),
        out_shape=jax.ShapeDtypeStruct((N, D), jnp.float32),
    )(feature_view, w1)

    BE = 12800
    u = pl.pallas_call(
        _proj_edges_body,
        grid=(E // BE,),
        in_specs=[
            pl.BlockSpec((A, BE), lambda i: (0, i)),
            pl.BlockSpec((T, BE), lambda i: (0, i)),
            pl.BlockSpec((A, D), lambda i: (0, 0)),
            pl.BlockSpec((T, D), lambda i: (0, 0)),
            pl.BlockSpec((1, D), lambda i: (0, 0)),
        ],
        out_specs=pl.BlockSpec((BE, D // 2), lambda i: (i, 0)),
        out_shape=jax.ShapeDtypeStruct((E, D // 2), jnp.int32),
    )(edge_attr.T, edge_time_emb.T, w2a, w2b, b2)

    mesh = plsc.VectorSubcoreMesh(core_axis_name="c", subcore_axis_name="s",
                                  num_cores=NC, num_subcores=NS)
    acc2 = pl.kernel(
        _sc_scatter_body,
        out_type=jax.ShapeDtypeStruct((NC * N, D), jnp.float32),
        mesh=mesh,
        scratch_types=[
            pltpu.VMEM((2, CH), jnp.int32),
            pltpu.VMEM((2, CH), jnp.int32),
            pltpu.VMEM((2, CH, D), jnp.float32),
            pltpu.VMEM((2, CH, D // 2), jnp.int32),
            pltpu.VMEM((ZROWS, D), jnp.float32),
            pltpu.VMEM_SHARED((N, D), jnp.float32),
            pltpu.SemaphoreType.DMA,
            pltpu.SemaphoreType.DMA,
            pltpu.SemaphoreType.DMA,
            pltpu.SemaphoreType.DMA,
            pltpu.SemaphoreType.DMA,
        ],
    )(p, u, src, dst)

    out = pl.pallas_call(
        _final_body,
        grid=(N // BN,),
        in_specs=[
            pl.BlockSpec((BN, D), lambda i: (i, 0)),
            pl.BlockSpec((BN, D), lambda i: (i + N // BN, 0)),
            pl.BlockSpec((BN, D), lambda i: (i, 0)),
            pl.BlockSpec((D, D), lambda i: (0, 0)),
            pl.BlockSpec((1, D), lambda i: (0, 0)),
            pl.BlockSpec((1, D), lambda i: (0, 0)),
            pl.BlockSpec((1, D), lambda i: (0, 0)),
        ],
        out_specs=pl.BlockSpec((BN, D), lambda i: (i, 0)),
        out_shape=jax.ShapeDtypeStruct((N, D), jnp.float32),
    )(acc2, acc2, boundary_condition, W_lin, b_lin.reshape(1, D),
      ln_gamma.reshape(1, D), ln_beta.reshape(1, D))

    return out


# speedup vs baseline: 2.3517x; 1.1077x over previous
import functools

import jax
import jax.numpy as jnp
from jax import lax
from jax.experimental import pallas as pl
from jax.experimental.pallas import tpu as pltpu
from jax.experimental.pallas import tpu_sc as plsc

N = 10000
E = 320000
D = 128
A = 16
T = 16

NC = 2
NS = 16
NW = NC * NS

CH = 80
NCHUNKS = E // CH
KPT = NCHUNKS // NW
ROWS_A = 624
TAIL_ROWS = N - NS * ROWS_A
ZROWS = 48


def _proj_nodes_body(fv_ref, w1_ref, p_ref):
    p_ref[...] = jnp.dot(fv_ref[...], w1_ref[...],
                         preferred_element_type=jnp.float32)


def _proj_edges_body(ea_ref, et_ref, w2_ref, b_ref, u_ref):
    dn = (((0,), (0,)), ((), ()))
    x = jnp.concatenate([ea_ref[...], et_ref[...]], axis=0)
    u = lax.dot_general(x, w2_ref[...], dn,
                        preferred_element_type=jnp.float32) + b_ref[...]
    lo = lax.bitcast_convert_type(u[:, :D // 2].astype(jnp.bfloat16),
                                  jnp.uint16).astype(jnp.uint32)
    hi = lax.bitcast_convert_type(u[:, D // 2:].astype(jnp.bfloat16),
                                  jnp.uint16).astype(jnp.uint32)
    u_ref[...] = lax.bitcast_convert_type((hi << 16) | lo, jnp.int32)


def _sc_scatter_body(p_hbm, u_hbm, src_hbm, dst_hbm, out_hbm,
                     src_v, dst_v, rows_v, u_v, zero_v, acc_sh,
                     sem_in0, sem_in1, sem_g0, sem_g1, sem_z):
    cid = lax.axis_index("c")
    sid = lax.axis_index("s")
    wid = sid * NC + cid
    sem_in = (sem_in0, sem_in1)
    sem_g = (sem_g0, sem_g1)

    def zero_buf(i, _):
        r = i // (D // 16)
        c = (i % (D // 16)) * 16
        zero_v[r, pl.ds(c, 16)] = jnp.zeros((16,), jnp.float32)
        return 0
    lax.fori_loop(0, ZROWS * (D // 16), zero_buf, 0, unroll=8)
    row0 = sid * ROWS_A
    zcopies = []
    for z in range(ROWS_A // ZROWS):
        zcopies.append(pltpu.async_copy(
            zero_v, acc_sh.at[pl.ds(row0 + z * ZROWS, ZROWS)], sem_z))

    @pl.when(sid == NS - 1)
    def _zero_tail():
        pltpu.async_copy(zero_v.at[pl.ds(0, TAIL_ROWS)],
                         acc_sh.at[pl.ds(NS * ROWS_A, TAIL_ROWS)], sem_z).wait()
    for zc in zcopies:
        zc.wait()
    plsc.subcore_barrier()

    def issue_in(k, b):
        base = (k * NW + wid) * CH
        pltpu.async_copy(src_hbm.at[pl.ds(base, CH)], src_v.at[b], sem_in[b])
        pltpu.async_copy(dst_hbm.at[pl.ds(base, CH)], dst_v.at[b], sem_in[b])
        pltpu.async_copy(u_hbm.at[pl.ds(base, CH)], u_v.at[b], sem_in[b])

    def wait_in_idx(b):
        pltpu.make_async_copy(src_hbm.at[pl.ds(0, CH)], src_v.at[b],
                              sem_in[b]).wait()
        pltpu.make_async_copy(dst_hbm.at[pl.ds(0, CH)], dst_v.at[b],
                              sem_in[b]).wait()

    def wait_in_u(b):
        pltpu.make_async_copy(u_hbm.at[pl.ds(0, CH)], u_v.at[b],
                              sem_in[b]).wait()

    def issue_gather(b):
        pltpu.async_copy(p_hbm.at[src_v.at[b]], rows_v.at[b], sem_g[b])

    def wait_gather(b):
        pltpu.make_async_copy(p_hbm.at[src_v.at[b]], rows_v.at[b],
                              sem_g[b]).wait()

    issue_in(0, 0)
    wait_in_idx(0)
    issue_gather(0)
    issue_in(1, 1)

    def phase(k, cur, nxt):
        wait_gather(cur)
        wait_in_u(cur)

        @pl.when(k + 1 < KPT)
        def _prefetch_gather():
            wait_in_idx(nxt)
            issue_gather(nxt)

        def fuse(r, _):
            for m in range(D // 32):
                w = u_v[cur, r, pl.ds(m * 16, 16)]
                wl = lax.bitcast_convert_type(lax.shift_left(w, 16),
                                              jnp.float32)
                wh = lax.bitcast_convert_type(
                    jnp.bitwise_and(w, jnp.int32(-65536)), jnp.float32)
                a = rows_v[cur, r, pl.ds(m * 16, 16)] + wl
                rows_v[cur, r, pl.ds(m * 16, 16)] = jnp.maximum(a, 0.0)
                b = rows_v[cur, r, pl.ds((m + D // 32) * 16, 16)] + wh
                rows_v[cur, r, pl.ds((m + D // 32) * 16, 16)] = (
                    jnp.maximum(b, 0.0))
            return 0
        lax.fori_loop(0, CH, fuse, 0)

        pltpu.sync_copy(rows_v.at[cur], acc_sh.at[dst_v.at[cur]], add=True)

        @pl.when(k + 2 < KPT)
        def _prefetch_in():
            issue_in(k + 2, cur)

    def do_pair(k2, _):
        phase(2 * k2, 0, 1)
        phase(2 * k2 + 1, 1, 0)
        return 0
    lax.fori_loop(0, KPT // 2, do_pair, 0)
    phase(jnp.int32(KPT - 1), 0, 1)

    plsc.subcore_barrier()

    pltpu.sync_copy(acc_sh.at[pl.ds(row0, ROWS_A)],
                    out_hbm.at[pl.ds(cid * N + row0, ROWS_A)])

    @pl.when(sid == NS - 1)
    def _drain_tail():
        pltpu.sync_copy(acc_sh.at[pl.ds(NS * ROWS_A, TAIL_ROWS)],
                        out_hbm.at[pl.ds(cid * N + NS * ROWS_A, TAIL_ROWS)])


def _final_body(a0_ref, a1_ref, bc_ref, wl_ref, bl_ref, g_ref, be_ref, o_ref):
    h = a0_ref[...] + a1_ref[...] + bc_ref[...]
    y = jnp.dot(h, wl_ref[...], preferred_element_type=jnp.float32) + bl_ref[...]
    mean = jnp.mean(y, axis=-1, keepdims=True)
    var = jnp.mean(jnp.square(y - mean), axis=-1, keepdims=True)
    yn = (y - mean) * lax.rsqrt(var + 1e-5) * g_ref[...] + be_ref[...]
    o_ref[...] = jnp.maximum(yn, 0.0)


def kernel(feature_view, edge_index, edge_attr, edge_time_emb,
           boundary_condition, W_msg, b_msg, W_lin, b_lin, ln_gamma, ln_beta):
    src = edge_index[0]
    dst = edge_index[1]
    w1 = W_msg[:D]
    w2a = W_msg[D:D + A]
    w2b = W_msg[D + A:]
    b2 = b_msg.reshape(1, D)

    BN = 1000
    p = pl.pallas_call(
        _proj_nodes_body,
        grid=(N // BN,),
        in_specs=[
            pl.BlockSpec((BN, D), lambda i: (i, 0)),
            pl.BlockSpec((D, D), lambda i: (0, 0)),
        ],
        out_specs=pl.BlockSpec((BN, D), lambda i: (i, 0)),
        out_shape=jax.ShapeDtypeStruct((N, D), jnp.float32),
    )(feature_view, w1)

    BE = 12800
    u = pl.pallas_call(
        _proj_edges_body,
        grid=(E // BE,),
        in_specs=[
            pl.BlockSpec((A, BE), lambda i: (0, i)),
            pl.BlockSpec((T, BE), lambda i: (0, i)),
            pl.BlockSpec((A + T, D), lambda i: (0, 0)),
            pl.BlockSpec((1, D), lambda i: (0, 0)),
        ],
        out_specs=pl.BlockSpec((BE, D // 2), lambda i: (i, 0)),
        out_shape=jax.ShapeDtypeStruct((E, D // 2), jnp.int32),
    )(edge_attr.T, edge_time_emb.T, W_msg[D:], b2)

    mesh = plsc.VectorSubcoreMesh(core_axis_name="c", subcore_axis_name="s",
                                  num_cores=NC, num_subcores=NS)
    acc2 = pl.kernel(
        _sc_scatter_body,
        out_type=jax.ShapeDtypeStruct((NC * N, D), jnp.float32),
        mesh=mesh,
        scratch_types=[
            pltpu.VMEM((2, CH), jnp.int32),
            pltpu.VMEM((2, CH), jnp.int32),
            pltpu.VMEM((2, CH, D), jnp.float32),
            pltpu.VMEM((2, CH, D // 2), jnp.int32),
            pltpu.VMEM((ZROWS, D), jnp.float32),
            pltpu.VMEM_SHARED((N, D), jnp.float32),
            pltpu.SemaphoreType.DMA,
            pltpu.SemaphoreType.DMA,
            pltpu.SemaphoreType.DMA,
            pltpu.SemaphoreType.DMA,
            pltpu.SemaphoreType.DMA,
        ],
    )(p, u, src, dst)

    out = pl.pallas_call(
        _final_body,
        grid=(N // BN,),
        in_specs=[
            pl.BlockSpec((BN, D), lambda i: (i, 0)),
            pl.BlockSpec((BN, D), lambda i: (i + N // BN, 0)),
            pl.BlockSpec((BN, D), lambda i: (i, 0)),
            pl.BlockSpec((D, D), lambda i: (0, 0)),
            pl.BlockSpec((1, D), lambda i: (0, 0)),
            pl.BlockSpec((1, D), lambda i: (0, 0)),
            pl.BlockSpec((1, D), lambda i: (0, 0)),
        ],
        out_specs=pl.BlockSpec((BN, D), lambda i: (i, 0)),
        out_shape=jax.ShapeDtypeStruct((N, D), jnp.float32),
    )(acc2, acc2, boundary_condition, W_lin, b_lin.reshape(1, D),
      ln_gamma.reshape(1, D), ln_beta.reshape(1, D))

    return out


# speedup vs baseline: 2.5348x; 1.0779x over previous
import functools

import jax
import jax.numpy as jnp
from jax import lax
from jax.experimental import pallas as pl
from jax.experimental.pallas import tpu as pltpu
from jax.experimental.pallas import tpu_sc as plsc

N = 10000
E = 320000
D = 128
A = 16
T = 16

NC = 2
NS = 16
NW = NC * NS

CH = 80
NCHUNKS = E // CH
KPT0 = 62
KPT1 = 63
SPLIT = KPT0 * NW * CH
ROWS_A = 624
TAIL_ROWS = N - NS * ROWS_A
ZROWS = 48


def _proj_nodes_body(fv_ref, w1_ref, p_ref):
    p_ref[...] = jnp.dot(fv_ref[...], w1_ref[...],
                         preferred_element_type=jnp.float32)


def _proj_edges_body(ea_ref, et_ref, w2_ref, b_ref, u_ref):
    dn = (((0,), (0,)), ((), ()))
    x = jnp.concatenate([ea_ref[...], et_ref[...]], axis=0)
    u = lax.dot_general(x, w2_ref[...], dn,
                        preferred_element_type=jnp.float32) + b_ref[...]
    lo = lax.bitcast_convert_type(u[:, :D // 2].astype(jnp.bfloat16),
                                  jnp.uint16).astype(jnp.uint32)
    hi = lax.bitcast_convert_type(u[:, D // 2:].astype(jnp.bfloat16),
                                  jnp.uint16).astype(jnp.uint32)
    u_ref[...] = lax.bitcast_convert_type((hi << 16) | lo, jnp.int32)


def _make_sc_body(chunk0, kpt):
    def body(p_hbm, u_hbm, src_hbm, dst_hbm, out_hbm,
             src_v, dst_v, rows_v, u_v, zero_v, acc_sh,
             sem_in0, sem_in1, sem_g0, sem_g1, sem_z):
        return _sc_scatter_impl(chunk0, kpt, p_hbm, u_hbm, src_hbm, dst_hbm,
                                out_hbm, src_v, dst_v, rows_v, u_v, zero_v,
                                acc_sh, sem_in0, sem_in1, sem_g0, sem_g1,
                                sem_z)
    return body


def _sc_scatter_impl(chunk0, kpt, p_hbm, u_hbm, src_hbm, dst_hbm, out_hbm,
                     src_v, dst_v, rows_v, u_v, zero_v, acc_sh,
                     sem_in0, sem_in1, sem_g0, sem_g1, sem_z):
    cid = lax.axis_index("c")
    sid = lax.axis_index("s")
    wid = sid * NC + cid
    sem_in = (sem_in0, sem_in1)
    sem_g = (sem_g0, sem_g1)

    def zero_buf(i, _):
        r = i // (D // 16)
        c = (i % (D // 16)) * 16
        zero_v[r, pl.ds(c, 16)] = jnp.zeros((16,), jnp.float32)
        return 0
    lax.fori_loop(0, ZROWS * (D // 16), zero_buf, 0, unroll=8)
    row0 = sid * ROWS_A
    zcopies = []
    for z in range(ROWS_A // ZROWS):
        zcopies.append(pltpu.async_copy(
            zero_v, acc_sh.at[pl.ds(row0 + z * ZROWS, ZROWS)], sem_z))

    @pl.when(sid == NS - 1)
    def _zero_tail():
        pltpu.async_copy(zero_v.at[pl.ds(0, TAIL_ROWS)],
                         acc_sh.at[pl.ds(NS * ROWS_A, TAIL_ROWS)], sem_z).wait()
    for zc in zcopies:
        zc.wait()
    plsc.subcore_barrier()

    def issue_in(k, b):
        lbase = (k * NW + wid) * CH
        gbase = chunk0 * CH + lbase
        pltpu.async_copy(src_hbm.at[pl.ds(gbase, CH)], src_v.at[b], sem_in[b])
        pltpu.async_copy(dst_hbm.at[pl.ds(gbase, CH)], dst_v.at[b], sem_in[b])
        pltpu.async_copy(u_hbm.at[pl.ds(lbase, CH)], u_v.at[b], sem_in[b])

    def wait_in_idx(b):
        pltpu.make_async_copy(src_hbm.at[pl.ds(0, CH)], src_v.at[b],
                              sem_in[b]).wait()
        pltpu.make_async_copy(dst_hbm.at[pl.ds(0, CH)], dst_v.at[b],
                              sem_in[b]).wait()

    def wait_in_u(b):
        pltpu.make_async_copy(u_hbm.at[pl.ds(0, CH)], u_v.at[b],
                              sem_in[b]).wait()

    def issue_gather(b):
        pltpu.async_copy(p_hbm.at[src_v.at[b]], rows_v.at[b], sem_g[b])

    def wait_gather(b):
        pltpu.make_async_copy(p_hbm.at[src_v.at[b]], rows_v.at[b],
                              sem_g[b]).wait()

    issue_in(0, 0)
    wait_in_idx(0)
    issue_gather(0)
    issue_in(1, 1)

    def phase(k, cur, nxt):
        wait_gather(cur)
        wait_in_u(cur)

        @pl.when(k + 1 < kpt)
        def _prefetch_gather():
            wait_in_idx(nxt)
            issue_gather(nxt)

        def fuse(r, _):
            for m in range(D // 32):
                w = u_v[cur, r, pl.ds(m * 16, 16)]
                wl = lax.bitcast_convert_type(lax.shift_left(w, 16),
                                              jnp.float32)
                wh = lax.bitcast_convert_type(
                    jnp.bitwise_and(w, jnp.int32(-65536)), jnp.float32)
                a = rows_v[cur, r, pl.ds(m * 16, 16)] + wl
                rows_v[cur, r, pl.ds(m * 16, 16)] = jnp.maximum(a, 0.0)
                b = rows_v[cur, r, pl.ds((m + D // 32) * 16, 16)] + wh
                rows_v[cur, r, pl.ds((m + D // 32) * 16, 16)] = (
                    jnp.maximum(b, 0.0))
            return 0
        lax.fori_loop(0, CH, fuse, 0)

        pltpu.sync_copy(rows_v.at[cur], acc_sh.at[dst_v.at[cur]], add=True)

        @pl.when(k + 2 < kpt)
        def _prefetch_in():
            issue_in(k + 2, cur)

    def do_pair(k2, _):
        phase(2 * k2, 0, 1)
        phase(2 * k2 + 1, 1, 0)
        return 0
    lax.fori_loop(0, kpt // 2, do_pair, 0)
    if kpt % 2:
        phase(jnp.int32(kpt - 1), 0, 1)

    plsc.subcore_barrier()

    pltpu.sync_copy(acc_sh.at[pl.ds(row0, ROWS_A)],
                    out_hbm.at[pl.ds(cid * N + row0, ROWS_A)])

    @pl.when(sid == NS - 1)
    def _drain_tail():
        pltpu.sync_copy(acc_sh.at[pl.ds(NS * ROWS_A, TAIL_ROWS)],
                        out_hbm.at[pl.ds(cid * N + NS * ROWS_A, TAIL_ROWS)])


def _final_body(a0_ref, a1_ref, a2_ref, a3_ref, bc_ref, wl_ref, bl_ref,
                g_ref, be_ref, o_ref):
    h = (a0_ref[...] + a1_ref[...] + a2_ref[...] + a3_ref[...]
         + bc_ref[...])
    y = jnp.dot(h, wl_ref[...], preferred_element_type=jnp.float32) + bl_ref[...]
    mean = jnp.mean(y, axis=-1, keepdims=True)
    var = jnp.mean(jnp.square(y - mean), axis=-1, keepdims=True)
    yn = (y - mean) * lax.rsqrt(var + 1e-5) * g_ref[...] + be_ref[...]
    o_ref[...] = jnp.maximum(yn, 0.0)


def kernel(feature_view, edge_index, edge_attr, edge_time_emb,
           boundary_condition, W_msg, b_msg, W_lin, b_lin, ln_gamma, ln_beta):
    src = edge_index[0]
    dst = edge_index[1]
    w1 = W_msg[:D]
    w2a = W_msg[D:D + A]
    w2b = W_msg[D + A:]
    b2 = b_msg.reshape(1, D)

    BN = 1000
    p = pl.pallas_call(
        _proj_nodes_body,
        grid=(N // BN,),
        in_specs=[
            pl.BlockSpec((BN, D), lambda i: (i, 0)),
            pl.BlockSpec((D, D), lambda i: (0, 0)),
        ],
        out_specs=pl.BlockSpec((BN, D), lambda i: (i, 0)),
        out_shape=jax.ShapeDtypeStruct((N, D), jnp.float32),
    )(feature_view, w1)

    BE = 2560
    halves = ((0, KPT0), (KPT0, KPT1))

    def proj_u(block0, nblocks):
        ne = nblocks * BE
        return pl.pallas_call(
            _proj_edges_body,
            grid=(nblocks,),
            in_specs=[
                pl.BlockSpec((A, BE), lambda i: (0, i + block0)),
                pl.BlockSpec((T, BE), lambda i: (0, i + block0)),
                pl.BlockSpec((A + T, D), lambda i: (0, 0)),
                pl.BlockSpec((1, D), lambda i: (0, 0)),
            ],
            out_specs=pl.BlockSpec((BE, D // 2), lambda i: (i, 0)),
            out_shape=jax.ShapeDtypeStruct((ne, D // 2), jnp.int32),
        )(edge_attr.T, edge_time_emb.T, W_msg[D:], b2)

    mesh = plsc.VectorSubcoreMesh(core_axis_name="c", subcore_axis_name="s",
                                  num_cores=NC, num_subcores=NS)

    def sc_half(chunk0, kpt, u_half):
        return pl.kernel(
            _make_sc_body(chunk0, kpt),
            out_type=jax.ShapeDtypeStruct((NC * N, D), jnp.float32),
            mesh=mesh,
            scratch_types=[
                pltpu.VMEM((2, CH), jnp.int32),
                pltpu.VMEM((2, CH), jnp.int32),
                pltpu.VMEM((2, CH, D), jnp.float32),
                pltpu.VMEM((2, CH, D // 2), jnp.int32),
                pltpu.VMEM((ZROWS, D), jnp.float32),
                pltpu.VMEM_SHARED((N, D), jnp.float32),
                pltpu.SemaphoreType.DMA,
                pltpu.SemaphoreType.DMA,
                pltpu.SemaphoreType.DMA,
                pltpu.SemaphoreType.DMA,
                pltpu.SemaphoreType.DMA,
            ],
        )(p, u_half, src, dst)

    u0 = proj_u(0, KPT0 * NW * CH // BE)
    acc_h0 = sc_half(0, KPT0, u0)
    u1 = proj_u(KPT0 * NW * CH // BE, KPT1 * NW * CH // BE)
    acc_h1 = sc_half(KPT0 * NW, KPT1, u1)

    out = pl.pallas_call(
        _final_body,
        grid=(N // BN,),
        in_specs=[
            pl.BlockSpec((BN, D), lambda i: (i, 0)),
            pl.BlockSpec((BN, D), lambda i: (i + N // BN, 0)),
            pl.BlockSpec((BN, D), lambda i: (i, 0)),
            pl.BlockSpec((BN, D), lambda i: (i + N // BN, 0)),
            pl.BlockSpec((BN, D), lambda i: (i, 0)),
            pl.BlockSpec((D, D), lambda i: (0, 0)),
            pl.BlockSpec((1, D), lambda i: (0, 0)),
            pl.BlockSpec((1, D), lambda i: (0, 0)),
            pl.BlockSpec((1, D), lambda i: (0, 0)),
        ],
        out_specs=pl.BlockSpec((BN, D), lambda i: (i, 0)),
        out_shape=jax.ShapeDtypeStruct((N, D), jnp.float32),
    )(acc_h0, acc_h0, acc_h1, acc_h1, boundary_condition, W_lin,
      b_lin.reshape(1, D), ln_gamma.reshape(1, D), ln_beta.reshape(1, D))

    return out
